# baseline (speedup 1.0000x reference)
import functools

import jax
import jax.numpy as jnp
from jax.experimental import pallas as pl
from jax.experimental.pallas import tpu as pltpu

_K = 32


def _f32_to_key(v):
    s = jax.lax.bitcast_convert_type(v, jnp.int32)
    return s ^ ((s >> 31) & jnp.int32(0x7FFFFFFF))


def _key_to_f32(k):
    s = k ^ ((k >> 31) & jnp.int32(0x7FFFFFFF))
    return jax.lax.bitcast_convert_type(s, jnp.float32)


def _sae_kernel(nh, k, cw, x_ref, we_ref, be_ref, wd_ref, bd_ref, out_ref,
                v_ref, cmax_ref, thr_ref):
    j = pl.program_id(1)
    ht = we_ref.shape[0]
    nc = ht // cw

    @pl.when(j < nh)
    def _encode():
        xt = x_ref[...] - bd_ref[...]
        pre = jax.lax.dot_general(
            xt, we_ref[...],
            dimension_numbers=(((1,), (1,)), ((), ())),
            preferred_element_type=jnp.float32,
        ) + be_ref[...]
        v_ref[:, pl.ds(j * ht, ht)] = pre
        cols = [jnp.max(pre[:, c * cw:(c + 1) * cw], axis=1, keepdims=True)
                for c in range(nc)]
        cmax_ref[j] = jnp.concatenate(cols, axis=1)

    @pl.when(j >= nh)
    def _decode():
        tb = v_ref.shape[0]

        @pl.when(j == nh)
        def _threshold():
            def step(carry, cnt):
                lo, hi, mid = carry
                ge = cnt >= k
                lo = jnp.where(ge, mid, lo)
                hi = jnp.where(ge, hi, mid - 1)
                return lo, hi

            def midpoint(lo, hi):
                return (lo >> 1) + (hi >> 1) + (lo & hi & 1) + ((lo ^ hi) & 1)

            cmax = jnp.concatenate([cmax_ref[i] for i in range(nh)], axis=1)
            ncx = cmax.shape[1]
            rmax = jnp.max(cmax, axis=1, keepdims=True)
            if ncx >= k:
                def body1(_, carry):
                    lo, hi = carry
                    mid = midpoint(lo, hi)
                    cnt = jnp.sum((cmax >= _key_to_f32(mid)).astype(jnp.int32),
                                  axis=1, keepdims=True)
                    return step((lo, hi, mid), cnt)

                lo0 = jnp.full((tb, 1), -0x80000000, dtype=jnp.int32)
                hi0 = _f32_to_key(rmax)
                lo0, _ = jax.lax.fori_loop(0, 32, body1, (lo0, hi0))
                hi0 = _f32_to_key(rmax)
            else:
                lo0 = jnp.full((tb, 1), -0x80000000, dtype=jnp.int32)
                hi0 = jnp.full((tb, 1), 0x7FFFFFFF, dtype=jnp.int32)

            def count_at(midf):
                cnt = jnp.zeros((tb, 1), dtype=jnp.int32)
                for c in range(nh):
                    chunk = v_ref[:, pl.ds(c * ht, ht)]
                    cnt = cnt + jnp.sum((chunk >= midf).astype(jnp.int32),
                                        axis=1, keepdims=True)
                return cnt

            kf = jnp.float32(k)
            clo0 = count_at(_key_to_f32(lo0)).astype(jnp.float32)
            chi0 = jnp.zeros((tb, 1), dtype=jnp.float32)

            def cond2(carry):
                it, lo, hi, clo, chi = carry
                return jnp.any(lo < hi)

            def body2(carry):
                it, lo, hi, clo, chi = carry
                t = (clo - kf) / jnp.maximum(clo - chi, jnp.float32(1.0))
                lof = lo.astype(jnp.float32)
                hif = hi.astype(jnp.float32)
                interp = jnp.clip((lof + (hif - lof) * t).astype(jnp.int32),
                                  lo + 1, hi)
                mid = jnp.where((it & 1) == 0, interp, midpoint(lo, hi))
                cnt = count_at(_key_to_f32(mid))
                cntf = cnt.astype(jnp.float32)
                ge = cnt >= k
                lo = jnp.where(ge, mid, lo)
                clo = jnp.where(ge, cntf, clo)
                hi = jnp.where(ge, hi, mid - 1)
                chi = jnp.where(ge, chi, cntf)
                return it + 1, lo, hi, clo, chi

            _, lo, _, _, _ = jax.lax.while_loop(
                cond2, body2, (jnp.int32(0), lo0, hi0, clo0, chi0))
            thr_ref[...] = _key_to_f32(lo)

        jd = j - nh
        chunk = v_ref[:, pl.ds(jd * ht, ht)]
        sparse = jnp.where(chunk >= thr_ref[...], chunk, jnp.float32(0.0))
        acc = jax.lax.dot_general(
            sparse, wd_ref[...],
            dimension_numbers=(((1,), (0,)), ((), ())),
            preferred_element_type=jnp.float32,
        )

        @pl.when(j == nh)
        def _init():
            out_ref[...] = acc + bd_ref[...]

        @pl.when(j > nh)
        def _acc():
            out_ref[...] = out_ref[...] + acc


def kernel(x, W_enc, b_enc, W_dec, b_dec):
    b, d = x.shape
    h = W_enc.shape[0]
    tile_b = min(512, b)
    nh = 8 if h % 8 == 0 else 1
    ht = h // nh
    cw = min(128, ht)

    be2 = b_enc.reshape(1, h)
    bd2 = b_dec.reshape(1, d)

    grid = (b // tile_b, 2 * nh)
    out = pl.pallas_call(
        functools.partial(_sae_kernel, nh, _K, cw),
        grid=grid,
        in_specs=[
            pl.BlockSpec((tile_b, d), lambda i, j: (i, 0)),
            pl.BlockSpec((ht, d), lambda i, j, _nh=nh: (jnp.minimum(j, _nh - 1), 0)),
            pl.BlockSpec((1, ht), lambda i, j, _nh=nh: (0, jnp.minimum(j, _nh - 1))),
            pl.BlockSpec((ht, d), lambda i, j, _nh=nh: (jnp.maximum(j - _nh, 0), 0)),
            pl.BlockSpec((1, d), lambda i, j: (0, 0)),
        ],
        out_specs=pl.BlockSpec((tile_b, d), lambda i, j: (i, 0)),
        out_shape=jax.ShapeDtypeStruct((b, d), jnp.float32),
        scratch_shapes=[pltpu.VMEM((tile_b, h), jnp.float32),
                        pltpu.VMEM((nh, tile_b, ht // cw), jnp.float32),
                        pltpu.VMEM((tile_b, 1), jnp.float32)],
        compiler_params=pltpu.CompilerParams(
            dimension_semantics=("arbitrary", "arbitrary"),
            vmem_limit_bytes=128 * 1024 * 1024,
        ),
    )(x, W_enc, be2, W_dec, bd2)
    return out


# speedup vs baseline: 1.8005x; 1.8005x over previous
import functools

import jax
import jax.numpy as jnp
from jax.experimental import pallas as pl
from jax.experimental.pallas import tpu as pltpu

_K = 32


def _f32_to_key(v):
    s = jax.lax.bitcast_convert_type(v, jnp.int32)
    return s ^ ((s >> 31) & jnp.int32(0x7FFFFFFF))


def _key_to_f32(k):
    s = k ^ ((k >> 31) & jnp.int32(0x7FFFFFFF))
    return jax.lax.bitcast_convert_type(s, jnp.float32)


def _sae_kernel(nh, k, cw, x_ref, we_ref, be_ref, wd_ref, bd_ref, out_ref,
                v_ref, cmax_ref, thr_ref):
    j = pl.program_id(1)
    ht = we_ref.shape[0]
    nc = ht // cw

    @pl.when(j < nh)
    def _encode():
        xt = x_ref[...] - bd_ref[...]
        pre = jax.lax.dot_general(
            xt, we_ref[...],
            dimension_numbers=(((1,), (1,)), ((), ())),
            preferred_element_type=jnp.float32,
        ) + be_ref[...]
        v_ref[:, pl.ds(j * ht, ht)] = pre
        cols = [jnp.max(pre[:, c * cw:(c + 1) * cw], axis=1, keepdims=True)
                for c in range(nc)]
        cmax_ref[j] = jnp.concatenate(cols, axis=1)

    @pl.when(j >= nh)
    def _decode():
        tb = v_ref.shape[0]

        @pl.when(j == nh)
        def _threshold():
            def step(carry, cnt):
                lo, hi, mid = carry
                ge = cnt >= k
                lo = jnp.where(ge, mid, lo)
                hi = jnp.where(ge, hi, mid - 1)
                return lo, hi

            def midpoint(lo, hi):
                return (lo >> 1) + (hi >> 1) + (lo & hi & 1) + ((lo ^ hi) & 1)

            cmax = jnp.concatenate([cmax_ref[i] for i in range(nh)], axis=1)
            ncx = cmax.shape[1]
            rmax = jnp.max(cmax, axis=1, keepdims=True)
            if ncx >= k:
                def body1(_, carry):
                    lo, hi = carry
                    mid = midpoint(lo, hi)
                    cnt = jnp.sum((cmax >= _key_to_f32(mid)).astype(jnp.int32),
                                  axis=1, keepdims=True)
                    return step((lo, hi, mid), cnt)

                lo0 = jnp.full((tb, 1), -0x80000000, dtype=jnp.int32)
                hi0 = _f32_to_key(rmax)
                lo0, _ = jax.lax.fori_loop(0, 32, body1, (lo0, hi0))
                hi0 = _f32_to_key(rmax)
            else:
                lo0 = jnp.full((tb, 1), -0x80000000, dtype=jnp.int32)
                hi0 = jnp.full((tb, 1), 0x7FFFFFFF, dtype=jnp.int32)

            def count_at(midf):
                cnt = jnp.zeros((tb, 1), dtype=jnp.int32)
                for c in range(nh):
                    chunk = v_ref[:, pl.ds(c * ht, ht)]
                    cnt = cnt + jnp.sum((chunk >= midf).astype(jnp.int32),
                                        axis=1, keepdims=True)
                return cnt

            kf = jnp.float32(k)
            clo0 = count_at(_key_to_f32(lo0)).astype(jnp.float32)
            chi0 = jnp.zeros((tb, 1), dtype=jnp.float32)

            def cond2(carry):
                it, lo, hi, clo, chi = carry
                return jnp.any((lo < hi) & (clo != kf))

            def body2(carry):
                it, lo, hi, clo, chi = carry
                t = (clo - kf) / jnp.maximum(clo - chi, jnp.float32(1.0))
                lof = lo.astype(jnp.float32)
                hif = hi.astype(jnp.float32)
                interpf = jnp.clip(lof + (hif - lof) * t,
                                   jnp.float32(-2.0e9), jnp.float32(2.0e9))
                interp = jnp.clip(interpf.astype(jnp.int32), lo + 1, hi)
                mid = jnp.where((it & 1) == 0, interp, midpoint(lo, hi))
                done = (clo == kf) | (lo >= hi)
                mid = jnp.where(done, lo, mid)
                cnt = count_at(_key_to_f32(mid))
                cntf = cnt.astype(jnp.float32)
                ge = cnt >= k
                lo = jnp.where(ge, mid, lo)
                clo = jnp.where(ge, cntf, clo)
                hi = jnp.where(ge, hi, mid - 1)
                chi = jnp.where(ge, chi, cntf)
                return it + 1, lo, hi, clo, chi

            _, lo, _, _, _ = jax.lax.while_loop(
                cond2, body2, (jnp.int32(0), lo0, hi0, clo0, chi0))
            thr_ref[...] = _key_to_f32(lo)

        jd = j - nh
        chunk = v_ref[:, pl.ds(jd * ht, ht)]
        sparse = jnp.where(chunk >= thr_ref[...], chunk, jnp.float32(0.0))
        acc = jax.lax.dot_general(
            sparse, wd_ref[...],
            dimension_numbers=(((1,), (0,)), ((), ())),
            preferred_element_type=jnp.float32,
        )

        @pl.when(j == nh)
        def _init():
            out_ref[...] = acc + bd_ref[...]

        @pl.when(j > nh)
        def _acc():
            out_ref[...] = out_ref[...] + acc


def kernel(x, W_enc, b_enc, W_dec, b_dec):
    b, d = x.shape
    h = W_enc.shape[0]
    tile_b = min(512, b)
    nh = 8 if h % 8 == 0 else 1
    ht = h // nh
    cw = min(128, ht)

    be2 = b_enc.reshape(1, h)
    bd2 = b_dec.reshape(1, d)

    grid = (b // tile_b, 2 * nh)
    out = pl.pallas_call(
        functools.partial(_sae_kernel, nh, _K, cw),
        grid=grid,
        in_specs=[
            pl.BlockSpec((tile_b, d), lambda i, j: (i, 0)),
            pl.BlockSpec((ht, d), lambda i, j, _nh=nh: (jnp.minimum(j, _nh - 1), 0)),
            pl.BlockSpec((1, ht), lambda i, j, _nh=nh: (0, jnp.minimum(j, _nh - 1))),
            pl.BlockSpec((ht, d), lambda i, j, _nh=nh: (jnp.maximum(j - _nh, 0), 0)),
            pl.BlockSpec((1, d), lambda i, j: (0, 0)),
        ],
        out_specs=pl.BlockSpec((tile_b, d), lambda i, j: (i, 0)),
        out_shape=jax.ShapeDtypeStruct((b, d), jnp.float32),
        scratch_shapes=[pltpu.VMEM((tile_b, h), jnp.float32),
                        pltpu.VMEM((nh, tile_b, ht // cw), jnp.float32),
                        pltpu.VMEM((tile_b, 1), jnp.float32)],
        compiler_params=pltpu.CompilerParams(
            dimension_semantics=("arbitrary", "arbitrary"),
            vmem_limit_bytes=128 * 1024 * 1024,
        ),
    )(x, W_enc, be2, W_dec, bd2)
    return out


# speedup vs baseline: 1.8018x; 1.0007x over previous
import functools

import jax
import jax.numpy as jnp
from jax.experimental import pallas as pl
from jax.experimental.pallas import tpu as pltpu

_K = 32


def _f32_to_key(v):
    s = jax.lax.bitcast_convert_type(v, jnp.int32)
    return s ^ ((s >> 31) & jnp.int32(0x7FFFFFFF))


def _key_to_f32(k):
    s = k ^ ((k >> 31) & jnp.int32(0x7FFFFFFF))
    return jax.lax.bitcast_convert_type(s, jnp.float32)


def _sae_kernel(nh, k, cw, x_ref, we_ref, be_ref, wd_ref, bd_ref, out_ref,
                v_ref, cmax_ref, thr_ref):
    j = pl.program_id(1)
    ht = we_ref.shape[0]
    nc = ht // cw

    @pl.when(j < nh)
    def _encode():
        xt = x_ref[...] - bd_ref[...]
        pre = jax.lax.dot_general(
            xt, we_ref[...],
            dimension_numbers=(((1,), (1,)), ((), ())),
            preferred_element_type=jnp.float32,
        ) + be_ref[...]
        v_ref[:, pl.ds(j * ht, ht)] = pre
        cols = [jnp.max(pre[:, c * cw:(c + 1) * cw], axis=1, keepdims=True)
                for c in range(nc)]
        cmax_ref[j] = jnp.concatenate(cols, axis=1)

    @pl.when(j >= nh)
    def _decode():
        tb = v_ref.shape[0]

        @pl.when(j == nh)
        def _threshold():
            def step(carry, cnt):
                lo, hi, mid = carry
                ge = cnt >= k
                lo = jnp.where(ge, mid, lo)
                hi = jnp.where(ge, hi, mid - 1)
                return lo, hi

            def midpoint(lo, hi):
                return (lo >> 1) + (hi >> 1) + (lo & hi & 1) + ((lo ^ hi) & 1)

            cmax = jnp.concatenate([cmax_ref[i] for i in range(nh)], axis=1)
            ncx = cmax.shape[1]
            rmax = jnp.max(cmax, axis=1, keepdims=True)
            if ncx >= k:
                def body1(_, carry):
                    lo, hi = carry
                    mid = midpoint(lo, hi)
                    cnt = jnp.sum((cmax >= _key_to_f32(mid)).astype(jnp.int32),
                                  axis=1, keepdims=True)
                    return step((lo, hi, mid), cnt)

                lo0 = jnp.full((tb, 1), -0x80000000, dtype=jnp.int32)
                hi0 = _f32_to_key(rmax)
                lo0, _ = jax.lax.fori_loop(0, 32, body1, (lo0, hi0))
                hi0 = _f32_to_key(rmax)
            else:
                lo0 = jnp.full((tb, 1), -0x80000000, dtype=jnp.int32)
                hi0 = jnp.full((tb, 1), 0x7FFFFFFF, dtype=jnp.int32)

            def count_at(midf):
                cnt = jnp.zeros((tb, 1), dtype=jnp.int32)
                for c in range(nh):
                    chunk = v_ref[:, pl.ds(c * ht, ht)]
                    cnt = cnt + jnp.sum((chunk >= midf).astype(jnp.int32),
                                        axis=1, keepdims=True)
                return cnt

            kf = jnp.float32(k)
            clo0 = count_at(_key_to_f32(lo0)).astype(jnp.float32)
            chi0 = jnp.zeros((tb, 1), dtype=jnp.float32)

            def cond2(carry):
                it, lo, hi, clo, chi = carry
                return jnp.any((lo < hi) & (clo != kf))

            def body2(carry):
                it, lo, hi, clo, chi = carry
                t = (clo - kf) / jnp.maximum(clo - chi, jnp.float32(1.0))
                lof = lo.astype(jnp.float32)
                hif = hi.astype(jnp.float32)
                interpf = jnp.clip(lof + (hif - lof) * t,
                                   jnp.float32(-2.0e9), jnp.float32(2.0e9))
                interp = jnp.clip(interpf.astype(jnp.int32), lo + 1, hi)
                mid = jnp.where((it & 1) == 0, interp, midpoint(lo, hi))
                done = (clo == kf) | (lo >= hi)
                mid = jnp.where(done, lo, mid)
                cnt = count_at(_key_to_f32(mid))
                cntf = cnt.astype(jnp.float32)
                ge = cnt >= k
                lo = jnp.where(ge, mid, lo)
                clo = jnp.where(ge, cntf, clo)
                hi = jnp.where(ge, hi, mid - 1)
                chi = jnp.where(ge, chi, cntf)
                return it + 1, lo, hi, clo, chi

            _, lo, _, _, _ = jax.lax.while_loop(
                cond2, body2, (jnp.int32(0), lo0, hi0, clo0, chi0))
            thr_ref[...] = _key_to_f32(lo)

        jd = j - nh
        chunk = v_ref[:, pl.ds(jd * ht, ht)]
        sparse = jnp.where(chunk >= thr_ref[...], chunk, jnp.float32(0.0))
        acc = jax.lax.dot_general(
            sparse, wd_ref[...],
            dimension_numbers=(((1,), (0,)), ((), ())),
            preferred_element_type=jnp.float32,
        )

        @pl.when(j == nh)
        def _init():
            out_ref[...] = acc + bd_ref[...]

        @pl.when(j > nh)
        def _acc():
            out_ref[...] = out_ref[...] + acc


def kernel(x, W_enc, b_enc, W_dec, b_dec):
    b, d = x.shape
    h = W_enc.shape[0]
    tile_b = min(512, b)
    nh = 8 if h % 8 == 0 else 1
    ht = h // nh
    cw = min(128, ht)

    be2 = b_enc.reshape(1, h)
    bd2 = b_dec.reshape(1, d)

    grid = (b // tile_b, 2 * nh)
    out = pl.pallas_call(
        functools.partial(_sae_kernel, nh, _K, cw),
        grid=grid,
        in_specs=[
            pl.BlockSpec((tile_b, d), lambda i, j: (i, 0)),
            pl.BlockSpec((ht, d), lambda i, j, _nh=nh: (jnp.minimum(j, _nh - 1), 0)),
            pl.BlockSpec((1, ht), lambda i, j, _nh=nh: (0, jnp.minimum(j, _nh - 1))),
            pl.BlockSpec((ht, d), lambda i, j, _nh=nh: (jnp.maximum(j - _nh, 0), 0)),
            pl.BlockSpec((1, d), lambda i, j: (0, 0)),
        ],
        out_specs=pl.BlockSpec((tile_b, d), lambda i, j: (i, 0)),
        out_shape=jax.ShapeDtypeStruct((b, d), jnp.float32),
        scratch_shapes=[pltpu.VMEM((tile_b, h), jnp.float32),
                        pltpu.VMEM((nh, tile_b, ht // cw), jnp.float32),
                        pltpu.VMEM((tile_b, 1), jnp.float32)],
        compiler_params=pltpu.CompilerParams(
            dimension_semantics=("parallel", "arbitrary"),
            vmem_limit_bytes=128 * 1024 * 1024,
        ),
    )(x, W_enc, be2, W_dec, bd2)
    return out
